# baseline (speedup 1.0000x reference)
import functools

import jax
import jax.numpy as jnp
from jax import lax
from jax.experimental import pallas as pl
from jax.experimental.pallas import tpu as pltpu
import jax.experimental.pallas.tpu_sc as plsc

N = 10000
D = 128
E = 320000

NC = 2
NS = 16
NW = NC * NS
EPW = E // NW
CHUNK = 80
NCHUNK = EPW // CHUNK
NPAD = 10240
RPT = NPAD // NS
CPT = NPAD // NS


def _seg_body(with_counts, h_hbm, e_hbm, part_hbm, *rest):
    if with_counts:
        (cnt_hbm, src_v, dst_v, dstc0, dstc1, rbuf0, rbuf1, acc_s, cnt_s,
         ones_v, cbuf, gsem0, gsem1, ssem0, ssem1) = rest
    else:
        (src_v, dst_v, dstc0, dstc1, rbuf0, rbuf1, acc_s,
         gsem0, gsem1, ssem0, ssem1) = rest
    c = lax.axis_index("c")
    s = lax.axis_index("s")
    w = c * NS + s

    pltpu.sync_copy(e_hbm.at[pl.ds(w * EPW, EPW)], src_v)
    pltpu.sync_copy(e_hbm.at[pl.ds(E + w * EPW, EPW)], dst_v)

    def _zrow(i, carry):
        def _zlane(j, carry2):
            rbuf0[i, pl.ds(j * 16, 16)] = jnp.zeros((16,), jnp.float32)
            return carry2
        return lax.fori_loop(0, D // 16, _zlane, carry)
    lax.fori_loop(0, CHUNK, _zrow, 0)
    for t in range(RPT // CHUNK):
        pltpu.sync_copy(rbuf0, acc_s.at[pl.ds(s * RPT + t * CHUNK, CHUNK), :])

    if with_counts:
        def _zc(i, carry):
            cbuf[pl.ds(i * 16, 16)] = jnp.zeros((16,), jnp.float32)
            return carry
        lax.fori_loop(0, CPT // 16, _zc, 0)
        pltpu.sync_copy(cbuf, cnt_s.at[pl.ds(s * CPT, CPT)])
        for j in range(CHUNK // 16):
            ones_v[pl.ds(j * 16, 16)] = jnp.ones((16,), jnp.float32)

    plsc.subcore_barrier()

    def _start(i, rb, sem):
        pltpu.async_copy(h_hbm.at[src_v.at[pl.ds(i * CHUNK, CHUNK)]], rb, sem)

    def _gwait(rb, sem):
        pltpu.make_async_copy(h_hbm.at[src_v.at[pl.ds(0, CHUNK)]], rb,
                              sem).wait()

    def _scatter(i, rb, dc, sem):
        _gwait(rb, gsem0 if rb is rbuf0 else gsem1)
        for k in range(CHUNK // 16):
            dc[pl.ds(k * 16, 16)] = dst_v[pl.ds(i * CHUNK + k * 16, 16)]
        pltpu.async_copy(rb, acc_s.at[dc], sem, add=True)
        if with_counts:
            pltpu.sync_copy(ones_v, cnt_s.at[dc], add=True)

    def _swait_refill(rb, dc, sem, prefetch_i):
        pltpu.make_async_copy(rb, acc_s.at[dc], sem).wait()

        @pl.when(prefetch_i < NCHUNK)
        def _():
            _start(prefetch_i, rb, gsem0 if rb is rbuf0 else gsem1)

    _start(0, rbuf0, gsem0)
    _start(1, rbuf1, gsem1)

    def _edge2(k, carry):
        i = 2 * k
        _scatter(i, rbuf0, dstc0, ssem0)
        _scatter(i + 1, rbuf1, dstc1, ssem1)
        _swait_refill(rbuf0, dstc0, ssem0, i + 2)
        _swait_refill(rbuf1, dstc1, ssem1, i + 3)
        return carry
    lax.fori_loop(0, NCHUNK // 2, _edge2, 0)
    _scatter(NCHUNK - 1, rbuf0, dstc0, ssem0)
    _swait_refill(rbuf0, dstc0, ssem0, NCHUNK)

    plsc.subcore_barrier()

    for t in range(RPT // CHUNK):
        pltpu.sync_copy(acc_s.at[pl.ds(s * RPT + t * CHUNK, CHUNK), :], rbuf0)
        pltpu.sync_copy(rbuf0,
                        part_hbm.at[c, pl.ds(s * RPT + t * CHUNK, CHUNK), :])
    if with_counts:
        pltpu.sync_copy(cnt_s.at[pl.ds(s * CPT, CPT)], cbuf)
        pltpu.sync_copy(cbuf, cnt_hbm.at[pl.ds(c * NPAD + s * CPT, CPT)])


def _make_seg(with_counts):
    out_type = [jax.ShapeDtypeStruct((NC, NPAD, D), jnp.float32)]
    scratch = [
        pltpu.VMEM((EPW,), jnp.int32),
        pltpu.VMEM((EPW,), jnp.int32),
        pltpu.VMEM((CHUNK,), jnp.int32),
        pltpu.VMEM((CHUNK,), jnp.int32),
        pltpu.VMEM((CHUNK, D), jnp.float32),
        pltpu.VMEM((CHUNK, D), jnp.float32),
        pltpu.VMEM_SHARED((NPAD, D), jnp.float32),
    ]
    if with_counts:
        out_type.append(jax.ShapeDtypeStruct((NC * NPAD,), jnp.float32))
        scratch += [
            pltpu.VMEM_SHARED((NPAD,), jnp.float32),
            pltpu.VMEM((CHUNK,), jnp.float32),
            pltpu.VMEM((CPT,), jnp.float32),
        ]
    scratch += [pltpu.SemaphoreType.DMA] * 4
    return pl.kernel(
        functools.partial(_seg_body, with_counts),
        out_type=out_type,
        mesh=plsc.VectorSubcoreMesh(core_axis_name="c", subcore_axis_name="s"),
        scratch_types=scratch,
    )


_seg_with_counts = _make_seg(True)
_seg_no_counts = _make_seg(False)


TBLK = 1024


def _t0_body(x_ref, w_ref, b_ref, o_ref):
    o_ref[...] = jnp.maximum(
        jnp.dot(x_ref[...], w_ref[...], preferred_element_type=jnp.float32)
        + b_ref[...], 0.0)


def _mean_agg(p0_ref, p1_ref, c0_ref, c1_ref):
    cnt = jnp.maximum(c0_ref[...] + c1_ref[...], 1.0)
    return (p0_ref[0] + p1_ref[0]) / cnt


def _layer_body(p0_ref, p1_ref, c0_ref, c1_ref, h_ref, wl_ref, bl_ref,
                wr_ref, o_ref):
    agg = _mean_agg(p0_ref, p1_ref, c0_ref, c1_ref)
    o_ref[...] = jnp.maximum(
        jnp.dot(agg, wl_ref[...], preferred_element_type=jnp.float32)
        + bl_ref[...]
        + jnp.dot(h_ref[...], wr_ref[...], preferred_element_type=jnp.float32),
        0.0)


def _layer_final_body(p0_ref, p1_ref, c0_ref, c1_ref, h_ref, wl_ref, bl_ref,
                      wr_ref, w1_ref, b1_ref, o_ref):
    agg = _mean_agg(p0_ref, p1_ref, c0_ref, c1_ref)
    h2 = jnp.maximum(
        jnp.dot(agg, wl_ref[...], preferred_element_type=jnp.float32)
        + bl_ref[...]
        + jnp.dot(h_ref[...], wr_ref[...], preferred_element_type=jnp.float32),
        0.0)
    o_ref[...] = (jnp.dot(h2, w1_ref[...], preferred_element_type=jnp.float32)
                  + b1_ref[...])


_row_spec = pl.BlockSpec((TBLK, D), lambda i: (i, 0))
_p0_spec = pl.BlockSpec((1, TBLK, D), lambda i: (0, i, 0))
_p1_spec = pl.BlockSpec((1, TBLK, D), lambda i: (1, i, 0))
_w_spec = pl.BlockSpec((D, D), lambda i: (0, 0))
_b_spec = pl.BlockSpec((1, D), lambda i: (0, 0))
_c_spec = pl.BlockSpec((TBLK, 1), lambda i: (i, 0))
_out_pad = jax.ShapeDtypeStruct((NPAD, D), jnp.float32)
_out_n = jax.ShapeDtypeStruct((N, D), jnp.float32)
_grid = (NPAD // TBLK,)

_t0 = pl.pallas_call(
    _t0_body, grid=_grid,
    in_specs=[_row_spec, _w_spec, _b_spec],
    out_specs=_row_spec, out_shape=_out_pad)

_layer = pl.pallas_call(
    _layer_body, grid=_grid,
    in_specs=[_p0_spec, _p1_spec, _c_spec, _c_spec, _row_spec,
              _w_spec, _b_spec, _w_spec],
    out_specs=_row_spec, out_shape=_out_pad)

_layer_final = pl.pallas_call(
    _layer_final_body, grid=_grid,
    in_specs=[_p0_spec, _p1_spec, _c_spec, _c_spec, _row_spec,
              _w_spec, _b_spec, _w_spec, _w_spec, _b_spec],
    out_specs=_row_spec, out_shape=_out_n)


def kernel(x, edge_index, W0, b0, Wl1, bl1, Wr1, Wl2, bl2, Wr2, W1, b1):
    eflat = edge_index.reshape(2 * E)
    b0r = b0.reshape(1, D)
    bl1r = bl1.reshape(1, D)
    bl2r = bl2.reshape(1, D)
    b1r = b1.reshape(1, D)

    h0 = _t0(x, W0, b0r)
    part1, cnt = _seg_with_counts(h0, eflat)
    c0 = cnt[:NPAD].reshape(NPAD, 1)
    c1 = cnt[NPAD:].reshape(NPAD, 1)
    h1 = _layer(part1, part1, c0, c1, h0, Wl1, bl1r, Wr1)
    (part2,) = _seg_no_counts(h1, eflat)
    out = _layer_final(part2, part2, c0, c1, h1, Wl2, bl2r, Wr2, W1, b1r)
    return out


# speedup vs baseline: 1.2230x; 1.2230x over previous
import functools

import jax
import jax.numpy as jnp
from jax import lax
from jax.experimental import pallas as pl
from jax.experimental.pallas import tpu as pltpu
import jax.experimental.pallas.tpu_sc as plsc

N = 10000
D = 128
E = 320000

NC = 2
NS = 16
NW = NC * NS
EPW = E // NW
CHUNK = 80
NCHUNK = EPW // CHUNK
NPAD = 10240
RPT = NPAD // NS
CPT = NPAD // NS


def _seg_body(with_counts, h_hbm, e_hbm, part_hbm, *rest):
    if with_counts:
        (cnt_hbm, src_v, dst_v, dstc0, dstc1, rbuf0, rbuf1, acc_s, cnt_s,
         ones_v, cbuf, gsem0, gsem1, ssem0, ssem1) = rest
    else:
        (src_v, dst_v, dstc0, dstc1, rbuf0, rbuf1, acc_s,
         gsem0, gsem1, ssem0, ssem1) = rest
    c = lax.axis_index("c")
    s = lax.axis_index("s")
    w = c * NS + s

    pltpu.sync_copy(e_hbm.at[pl.ds(w * EPW, EPW)], src_v)
    pltpu.sync_copy(e_hbm.at[pl.ds(E + w * EPW, EPW)], dst_v)

    def _zrow(i, carry):
        def _zlane(j, carry2):
            rbuf0[i, pl.ds(j * 16, 16)] = jnp.zeros((16,), jnp.float32)
            return carry2
        return lax.fori_loop(0, D // 16, _zlane, carry)
    lax.fori_loop(0, CHUNK, _zrow, 0)
    for t in range(RPT // CHUNK):
        pltpu.sync_copy(rbuf0, acc_s.at[pl.ds(s * RPT + t * CHUNK, CHUNK), :])

    if with_counts:
        def _zc(i, carry):
            cbuf[pl.ds(i * 16, 16)] = jnp.zeros((16,), jnp.float32)
            return carry
        lax.fori_loop(0, CPT // 16, _zc, 0)
        pltpu.sync_copy(cbuf, cnt_s.at[pl.ds(s * CPT, CPT)])
        for j in range(CHUNK // 16):
            ones_v[pl.ds(j * 16, 16)] = jnp.ones((16,), jnp.float32)

    plsc.subcore_barrier()

    def _start(i, rb, sem):
        pltpu.async_copy(h_hbm.at[src_v.at[pl.ds(i * CHUNK, CHUNK)]], rb, sem)

    def _gwait(rb, sem):
        pltpu.make_async_copy(h_hbm.at[src_v.at[pl.ds(0, CHUNK)]], rb,
                              sem).wait()

    def _consume(i, rb, dc, gsem, prefetch_i):
        _gwait(rb, gsem)
        for k in range(CHUNK // 16):
            dc[pl.ds(k * 16, 16)] = dst_v[pl.ds(i * CHUNK + k * 16, 16)]
        pltpu.sync_copy(rb, acc_s.at[dc], add=True)
        if with_counts:
            pltpu.sync_copy(ones_v, cnt_s.at[dc], add=True)

        @pl.when(prefetch_i < NCHUNK)
        def _():
            _start(prefetch_i, rb, gsem)

    _start(0, rbuf0, gsem0)
    _start(1, rbuf1, gsem1)

    def _edge2(k, carry):
        i = 2 * k
        _consume(i, rbuf0, dstc0, gsem0, i + 2)
        _consume(i + 1, rbuf1, dstc1, gsem1, i + 3)
        return carry
    lax.fori_loop(0, NCHUNK // 2, _edge2, 0)
    _consume(NCHUNK - 1, rbuf0, dstc0, gsem0, NCHUNK)

    plsc.subcore_barrier()

    for t in range(RPT // CHUNK):
        pltpu.sync_copy(acc_s.at[pl.ds(s * RPT + t * CHUNK, CHUNK), :], rbuf0)
        pltpu.sync_copy(rbuf0,
                        part_hbm.at[c, pl.ds(s * RPT + t * CHUNK, CHUNK), :])
    if with_counts:
        pltpu.sync_copy(cnt_s.at[pl.ds(s * CPT, CPT)], cbuf)
        pltpu.sync_copy(cbuf, cnt_hbm.at[pl.ds(c * NPAD + s * CPT, CPT)])


def _make_seg(with_counts):
    out_type = [jax.ShapeDtypeStruct((NC, NPAD, D), jnp.float32)]
    scratch = [
        pltpu.VMEM((EPW,), jnp.int32),
        pltpu.VMEM((EPW,), jnp.int32),
        pltpu.VMEM((CHUNK,), jnp.int32),
        pltpu.VMEM((CHUNK,), jnp.int32),
        pltpu.VMEM((CHUNK, D), jnp.float32),
        pltpu.VMEM((CHUNK, D), jnp.float32),
        pltpu.VMEM_SHARED((NPAD, D), jnp.float32),
    ]
    if with_counts:
        out_type.append(jax.ShapeDtypeStruct((NC * NPAD,), jnp.float32))
        scratch += [
            pltpu.VMEM_SHARED((NPAD,), jnp.float32),
            pltpu.VMEM((CHUNK,), jnp.float32),
            pltpu.VMEM((CPT,), jnp.float32),
        ]
    scratch += [pltpu.SemaphoreType.DMA] * 4
    return pl.kernel(
        functools.partial(_seg_body, with_counts),
        out_type=out_type,
        mesh=plsc.VectorSubcoreMesh(core_axis_name="c", subcore_axis_name="s"),
        scratch_types=scratch,
    )


_seg_with_counts = _make_seg(True)
_seg_no_counts = _make_seg(False)


TBLK = 1024


def _t0_body(x_ref, w_ref, b_ref, o_ref):
    o_ref[...] = jnp.maximum(
        jnp.dot(x_ref[...], w_ref[...], preferred_element_type=jnp.float32)
        + b_ref[...], 0.0)


def _mean_agg(p0_ref, p1_ref, c0_ref, c1_ref):
    cnt = jnp.maximum(c0_ref[...] + c1_ref[...], 1.0)
    return (p0_ref[0] + p1_ref[0]) / cnt


def _layer_body(p0_ref, p1_ref, c0_ref, c1_ref, h_ref, wl_ref, bl_ref,
                wr_ref, o_ref):
    agg = _mean_agg(p0_ref, p1_ref, c0_ref, c1_ref)
    o_ref[...] = jnp.maximum(
        jnp.dot(agg, wl_ref[...], preferred_element_type=jnp.float32)
        + bl_ref[...]
        + jnp.dot(h_ref[...], wr_ref[...], preferred_element_type=jnp.float32),
        0.0)


def _layer_final_body(p0_ref, p1_ref, c0_ref, c1_ref, h_ref, wl_ref, bl_ref,
                      wr_ref, w1_ref, b1_ref, o_ref):
    agg = _mean_agg(p0_ref, p1_ref, c0_ref, c1_ref)
    h2 = jnp.maximum(
        jnp.dot(agg, wl_ref[...], preferred_element_type=jnp.float32)
        + bl_ref[...]
        + jnp.dot(h_ref[...], wr_ref[...], preferred_element_type=jnp.float32),
        0.0)
    o_ref[...] = (jnp.dot(h2, w1_ref[...], preferred_element_type=jnp.float32)
                  + b1_ref[...])


_row_spec = pl.BlockSpec((TBLK, D), lambda i: (i, 0))
_p0_spec = pl.BlockSpec((1, TBLK, D), lambda i: (0, i, 0))
_p1_spec = pl.BlockSpec((1, TBLK, D), lambda i: (1, i, 0))
_w_spec = pl.BlockSpec((D, D), lambda i: (0, 0))
_b_spec = pl.BlockSpec((1, D), lambda i: (0, 0))
_c_spec = pl.BlockSpec((TBLK, 1), lambda i: (i, 0))
_out_pad = jax.ShapeDtypeStruct((NPAD, D), jnp.float32)
_out_n = jax.ShapeDtypeStruct((N, D), jnp.float32)
_grid = (NPAD // TBLK,)

_t0 = pl.pallas_call(
    _t0_body, grid=_grid,
    in_specs=[_row_spec, _w_spec, _b_spec],
    out_specs=_row_spec, out_shape=_out_pad)

_layer = pl.pallas_call(
    _layer_body, grid=_grid,
    in_specs=[_p0_spec, _p1_spec, _c_spec, _c_spec, _row_spec,
              _w_spec, _b_spec, _w_spec],
    out_specs=_row_spec, out_shape=_out_pad)

_layer_final = pl.pallas_call(
    _layer_final_body, grid=_grid,
    in_specs=[_p0_spec, _p1_spec, _c_spec, _c_spec, _row_spec,
              _w_spec, _b_spec, _w_spec, _w_spec, _b_spec],
    out_specs=_row_spec, out_shape=_out_n)


def kernel(x, edge_index, W0, b0, Wl1, bl1, Wr1, Wl2, bl2, Wr2, W1, b1):
    eflat = edge_index.reshape(2 * E)
    b0r = b0.reshape(1, D)
    bl1r = bl1.reshape(1, D)
    bl2r = bl2.reshape(1, D)
    b1r = b1.reshape(1, D)

    h0 = _t0(x, W0, b0r)
    part1, cnt = _seg_with_counts(h0, eflat)
    c0 = cnt[:NPAD].reshape(NPAD, 1)
    c1 = cnt[NPAD:].reshape(NPAD, 1)
    h1 = _layer(part1, part1, c0, c1, h0, Wl1, bl1r, Wr1)
    (part2,) = _seg_no_counts(h1, eflat)
    out = _layer_final(part2, part2, c0, c1, h1, Wl2, bl2r, Wr2, W1, b1r)
    return out


# speedup vs baseline: 1.2644x; 1.0339x over previous
import functools

import jax
import jax.numpy as jnp
from jax import lax
from jax.experimental import pallas as pl
from jax.experimental.pallas import tpu as pltpu
import jax.experimental.pallas.tpu_sc as plsc

N = 10000
D = 128
E = 320000

NC = 2
NS = 16
NW = NC * NS
EPW = E // NW
CHUNK = 80
NCHUNK = EPW // CHUNK
NPAD = 10240
RPT = NPAD // NS
CPT = NPAD // NS


def _seg_body(with_counts, h_hbm, e_hbm, part_hbm, *rest):
    if with_counts:
        (cnt_hbm, src_v, dst_v, dstc0, dstc1, rbuf0, rbuf1, acc_s, cnt_s,
         ones_v, cbuf, gsem0, gsem1, ssem0, ssem1) = rest
    else:
        (src_v, dst_v, dstc0, dstc1, rbuf0, rbuf1, acc_s,
         gsem0, gsem1, ssem0, ssem1) = rest
    c = lax.axis_index("c")
    s = lax.axis_index("s")
    w = c * NS + s

    pltpu.sync_copy(e_hbm.at[pl.ds(w * EPW, EPW)], src_v)
    pltpu.sync_copy(e_hbm.at[pl.ds(E + w * EPW, EPW)], dst_v)

    def _zrow(i, carry):
        def _zlane(j, carry2):
            rbuf0[i, pl.ds(j * 16, 16)] = jnp.zeros((16,), jnp.float32)
            return carry2
        return lax.fori_loop(0, D // 16, _zlane, carry)
    lax.fori_loop(0, CHUNK, _zrow, 0)
    for t in range(RPT // CHUNK):
        pltpu.sync_copy(rbuf0, acc_s.at[pl.ds(s * RPT + t * CHUNK, CHUNK), :])

    if with_counts:
        def _zc(i, carry):
            cbuf[pl.ds(i * 16, 16)] = jnp.zeros((16,), jnp.float32)
            return carry
        lax.fori_loop(0, CPT // 16, _zc, 0)
        pltpu.sync_copy(cbuf, cnt_s.at[pl.ds(s * CPT, CPT)])
        for j in range(CHUNK // 16):
            ones_v[pl.ds(j * 16, 16)] = jnp.ones((16,), jnp.float32)

    plsc.subcore_barrier()

    def _start(i, rb, sem):
        pltpu.async_copy(h_hbm.at[src_v.at[pl.ds(i * CHUNK, CHUNK)]], rb, sem)

    def _gwait(rb, sem):
        pltpu.make_async_copy(h_hbm.at[src_v.at[pl.ds(0, CHUNK)]], rb,
                              sem).wait()

    def _consume(i, rb, dc, gsem, prefetch_i):
        _gwait(rb, gsem)
        for k in range(CHUNK // 16):
            dc[pl.ds(k * 16, 16)] = dst_v[pl.ds(i * CHUNK + k * 16, 16)]
        pltpu.sync_copy(rb, acc_s.at[dc], add=True)
        if with_counts:
            pltpu.sync_copy(ones_v, cnt_s.at[dc], add=True)

        @pl.when(prefetch_i < NCHUNK)
        def _():
            _start(prefetch_i, rb, gsem)

    _start(0, rbuf0, gsem0)
    _start(1, rbuf1, gsem1)

    def _edge2(k, carry):
        i = 2 * k
        _consume(i, rbuf0, dstc0, gsem0, i + 2)
        _consume(i + 1, rbuf1, dstc1, gsem1, i + 3)
        return carry
    lax.fori_loop(0, NCHUNK // 2, _edge2, 0)
    _consume(NCHUNK - 1, rbuf0, dstc0, gsem0, NCHUNK)

    plsc.subcore_barrier()

    for t in range(RPT // CHUNK):
        pltpu.sync_copy(acc_s.at[pl.ds(s * RPT + t * CHUNK, CHUNK), :], rbuf0)
        pltpu.sync_copy(rbuf0,
                        part_hbm.at[c, pl.ds(s * RPT + t * CHUNK, CHUNK), :])
    if with_counts:
        pltpu.sync_copy(cnt_s.at[pl.ds(s * CPT, CPT)], cbuf)
        pltpu.sync_copy(cbuf, cnt_hbm.at[pl.ds(c * NPAD + s * CPT, CPT)])


def _make_seg(with_counts):
    out_type = [jax.ShapeDtypeStruct((NC, NPAD, D), jnp.float32)]
    scratch = [
        pltpu.VMEM((EPW,), jnp.int32),
        pltpu.VMEM((EPW,), jnp.int32),
        pltpu.VMEM((CHUNK,), jnp.int32),
        pltpu.VMEM((CHUNK,), jnp.int32),
        pltpu.VMEM((CHUNK, D), jnp.float32),
        pltpu.VMEM((CHUNK, D), jnp.float32),
        pltpu.VMEM_SHARED((NPAD, D), jnp.float32),
    ]
    if with_counts:
        out_type.append(jax.ShapeDtypeStruct((NC * NPAD,), jnp.float32))
        scratch += [
            pltpu.VMEM_SHARED((NPAD,), jnp.float32),
            pltpu.VMEM((CHUNK,), jnp.float32),
            pltpu.VMEM((CPT,), jnp.float32),
        ]
    scratch += [pltpu.SemaphoreType.DMA] * 4
    return pl.kernel(
        functools.partial(_seg_body, with_counts),
        out_type=out_type,
        mesh=plsc.VectorSubcoreMesh(core_axis_name="c", subcore_axis_name="s"),
        scratch_types=scratch,
    )


_seg_with_counts = _make_seg(True)
_seg_no_counts = _make_seg(False)


TBLK = 1024


def _t0_body(x_ref, w_ref, b_ref, o_ref):
    o_ref[...] = jnp.maximum(
        jnp.dot(x_ref[...], w_ref[...], preferred_element_type=jnp.float32)
        + b_ref[...], 0.0)


def _mean_agg(p0_ref, p1_ref, c_ref):
    cs = c_ref[0] + c_ref[1]
    cols = []
    for j in range(TBLK // 128):
        m = jnp.broadcast_to(cs[j][None, :], (128, 128))
        cols.append(m.T)
    cnt = jnp.maximum(jnp.concatenate(cols, axis=0), 1.0)
    return (p0_ref[0] + p1_ref[0]) / cnt


def _layer_body(p0_ref, p1_ref, c_ref, h_ref, wl_ref, bl_ref,
                wr_ref, o_ref):
    agg = _mean_agg(p0_ref, p1_ref, c_ref)
    o_ref[...] = jnp.maximum(
        jnp.dot(agg, wl_ref[...], preferred_element_type=jnp.float32)
        + bl_ref[...]
        + jnp.dot(h_ref[...], wr_ref[...], preferred_element_type=jnp.float32),
        0.0)


def _layer_final_body(p0_ref, p1_ref, c_ref, h_ref, wl_ref, bl_ref,
                      wr_ref, w1_ref, b1_ref, o_ref):
    agg = _mean_agg(p0_ref, p1_ref, c_ref)
    h2 = jnp.maximum(
        jnp.dot(agg, wl_ref[...], preferred_element_type=jnp.float32)
        + bl_ref[...]
        + jnp.dot(h_ref[...], wr_ref[...], preferred_element_type=jnp.float32),
        0.0)
    o_ref[...] = (jnp.dot(h2, w1_ref[...], preferred_element_type=jnp.float32)
                  + b1_ref[...])


_row_spec = pl.BlockSpec((TBLK, D), lambda i: (i, 0))
_p0_spec = pl.BlockSpec((1, TBLK, D), lambda i: (0, i, 0))
_p1_spec = pl.BlockSpec((1, TBLK, D), lambda i: (1, i, 0))
_w_spec = pl.BlockSpec((D, D), lambda i: (0, 0))
_b_spec = pl.BlockSpec((1, D), lambda i: (0, 0))
_c_spec = pl.BlockSpec((2, TBLK // 128, 128), lambda i: (0, i, 0))
_out_pad = jax.ShapeDtypeStruct((NPAD, D), jnp.float32)
_out_n = jax.ShapeDtypeStruct((N, D), jnp.float32)
_grid = (NPAD // TBLK,)

_t0 = pl.pallas_call(
    _t0_body, grid=_grid,
    in_specs=[_row_spec, _w_spec, _b_spec],
    out_specs=_row_spec, out_shape=_out_pad)

_layer = pl.pallas_call(
    _layer_body, grid=_grid,
    in_specs=[_p0_spec, _p1_spec, _c_spec, _row_spec,
              _w_spec, _b_spec, _w_spec],
    out_specs=_row_spec, out_shape=_out_pad)

_layer_final = pl.pallas_call(
    _layer_final_body, grid=_grid,
    in_specs=[_p0_spec, _p1_spec, _c_spec, _row_spec,
              _w_spec, _b_spec, _w_spec, _w_spec, _b_spec],
    out_specs=_row_spec, out_shape=_out_n)


def kernel(x, edge_index, W0, b0, Wl1, bl1, Wr1, Wl2, bl2, Wr2, W1, b1):
    eflat = edge_index.reshape(2 * E)
    b0r = b0.reshape(1, D)
    bl1r = bl1.reshape(1, D)
    bl2r = bl2.reshape(1, D)
    b1r = b1.reshape(1, D)

    h0 = _t0(x, W0, b0r)
    part1, cnt = _seg_with_counts(h0, eflat)
    cnt3 = cnt.reshape(NC, NPAD // 128, 128)
    h1 = _layer(part1, part1, cnt3, h0, Wl1, bl1r, Wr1)
    (part2,) = _seg_no_counts(h1, eflat)
    out = _layer_final(part2, part2, cnt3, h1, Wl2, bl2r, Wr2, W1, b1r)
    return out


# speedup vs baseline: 1.3167x; 1.0414x over previous
import functools

import jax
import jax.numpy as jnp
from jax import lax
from jax.experimental import pallas as pl
from jax.experimental.pallas import tpu as pltpu
import jax.experimental.pallas.tpu_sc as plsc

N = 10000
D = 128
E = 320000

NC = 2
NS = 16
NW = NC * NS
EPW = E // NW
CHUNK = 96
NCHUNK = 104
TAIL = EPW - NCHUNK * CHUNK
NPAD = 10240
RPT = NPAD // NS
CPT = NPAD // NS


def _seg_body(with_counts, h_hbm, e_hbm, part_hbm, *rest):
    if with_counts:
        (cnt_hbm, src_v, dst_v, dstc0, dstc1, dstct, rbuf0, rbuf1, tbuf,
         acc_s, cnt_s, ones_v, ones_t, cbuf, gsem0, gsem1) = rest
    else:
        (src_v, dst_v, dstc0, dstc1, dstct, rbuf0, rbuf1, tbuf, acc_s,
         gsem0, gsem1) = rest
    c = lax.axis_index("c")
    s = lax.axis_index("s")
    w = c * NS + s

    pltpu.sync_copy(e_hbm.at[pl.ds(w * EPW, EPW)], src_v)
    pltpu.sync_copy(e_hbm.at[pl.ds(E + w * EPW, EPW)], dst_v)

    def _zrow(i, carry):
        def _zlane(j, carry2):
            rbuf0[i, pl.ds(j * 16, 16)] = jnp.zeros((16,), jnp.float32)
            return carry2
        return lax.fori_loop(0, D // 16, _zlane, carry)
    lax.fori_loop(0, CHUNK, _zrow, 0)
    zoffs = [(t * CHUNK, CHUNK) for t in range(RPT // CHUNK)]
    zoffs.append(((RPT // CHUNK) * CHUNK, RPT - (RPT // CHUNK) * CHUNK))
    for o, sz in zoffs:
        pltpu.sync_copy(rbuf0.at[pl.ds(0, sz), :],
                        acc_s.at[pl.ds(s * RPT + o, sz), :])

    if with_counts:
        def _zc(i, carry):
            cbuf[pl.ds(i * 16, 16)] = jnp.zeros((16,), jnp.float32)
            return carry
        lax.fori_loop(0, CPT // 16, _zc, 0)
        pltpu.sync_copy(cbuf, cnt_s.at[pl.ds(s * CPT, CPT)])
        for j in range(CHUNK // 16):
            ones_v[pl.ds(j * 16, 16)] = jnp.ones((16,), jnp.float32)
        ones_t[pl.ds(0, 16)] = jnp.ones((16,), jnp.float32)

    plsc.subcore_barrier()

    def _start(i, rb, sem):
        pltpu.async_copy(h_hbm.at[src_v.at[pl.ds(i * CHUNK, CHUNK)]], rb, sem)

    def _gwait(rb, sem):
        pltpu.make_async_copy(h_hbm.at[src_v.at[pl.ds(0, CHUNK)]], rb,
                              sem).wait()

    def _consume(i, rb, dc, gsem, prefetch_i):
        _gwait(rb, gsem)
        for k in range(CHUNK // 16):
            dc[pl.ds(k * 16, 16)] = dst_v[pl.ds(i * CHUNK + k * 16, 16)]
        pltpu.sync_copy(rb, acc_s.at[dc], add=True)
        if with_counts:
            pltpu.sync_copy(ones_v, cnt_s.at[dc], add=True)

        @pl.when(prefetch_i < NCHUNK)
        def _():
            _start(prefetch_i, rb, gsem)

    _start(0, rbuf0, gsem0)
    _start(1, rbuf1, gsem1)

    def _edge2(k, carry):
        i = 2 * k
        _consume(i, rbuf0, dstc0, gsem0, i + 2)
        _consume(i + 1, rbuf1, dstc1, gsem1, i + 3)
        return carry
    lax.fori_loop(0, NCHUNK // 2, _edge2, 0)

    pltpu.async_copy(h_hbm.at[src_v.at[pl.ds(NCHUNK * CHUNK, TAIL)]], tbuf,
                     gsem0)
    dstct[pl.ds(0, TAIL)] = dst_v[pl.ds(NCHUNK * CHUNK, TAIL)]
    pltpu.make_async_copy(h_hbm.at[src_v.at[pl.ds(0, TAIL)]], tbuf,
                          gsem0).wait()
    pltpu.sync_copy(tbuf, acc_s.at[dstct], add=True)
    if with_counts:
        pltpu.sync_copy(ones_t, cnt_s.at[dstct], add=True)

    plsc.subcore_barrier()

    for o, sz in zoffs:
        pltpu.sync_copy(acc_s.at[pl.ds(s * RPT + o, sz), :],
                        rbuf0.at[pl.ds(0, sz), :])
        pltpu.sync_copy(rbuf0.at[pl.ds(0, sz), :],
                        part_hbm.at[c, pl.ds(s * RPT + o, sz), :])
    if with_counts:
        pltpu.sync_copy(cnt_s.at[pl.ds(s * CPT, CPT)], cbuf)
        pltpu.sync_copy(cbuf, cnt_hbm.at[pl.ds(c * NPAD + s * CPT, CPT)])


def _make_seg(with_counts):
    out_type = [jax.ShapeDtypeStruct((NC, NPAD, D), jnp.float32)]
    scratch = [
        pltpu.VMEM((EPW,), jnp.int32),
        pltpu.VMEM((EPW,), jnp.int32),
        pltpu.VMEM((CHUNK,), jnp.int32),
        pltpu.VMEM((CHUNK,), jnp.int32),
        pltpu.VMEM((TAIL,), jnp.int32),
        pltpu.VMEM((CHUNK, D), jnp.float32),
        pltpu.VMEM((CHUNK, D), jnp.float32),
        pltpu.VMEM((TAIL, D), jnp.float32),
        pltpu.VMEM_SHARED((NPAD, D), jnp.float32),
    ]
    if with_counts:
        out_type.append(jax.ShapeDtypeStruct((NC * NPAD,), jnp.float32))
        scratch += [
            pltpu.VMEM_SHARED((NPAD,), jnp.float32),
            pltpu.VMEM((CHUNK,), jnp.float32),
            pltpu.VMEM((TAIL,), jnp.float32),
            pltpu.VMEM((CPT,), jnp.float32),
        ]
    scratch += [pltpu.SemaphoreType.DMA] * 2
    return pl.kernel(
        functools.partial(_seg_body, with_counts),
        out_type=out_type,
        mesh=plsc.VectorSubcoreMesh(core_axis_name="c", subcore_axis_name="s"),
        scratch_types=scratch,
    )


_seg_with_counts = _make_seg(True)
_seg_no_counts = _make_seg(False)


TBLK = 1024


def _t0_body(x_ref, w_ref, b_ref, o_ref):
    o_ref[...] = jnp.maximum(
        jnp.dot(x_ref[...], w_ref[...], preferred_element_type=jnp.float32)
        + b_ref[...], 0.0)


def _mean_agg(p0_ref, p1_ref, c_ref):
    cs = c_ref[0] + c_ref[1]
    cols = []
    for j in range(TBLK // 128):
        m = jnp.broadcast_to(cs[j][None, :], (128, 128))
        cols.append(m.T)
    cnt = jnp.maximum(jnp.concatenate(cols, axis=0), 1.0)
    return (p0_ref[0] + p1_ref[0]) / cnt


def _layer_body(p0_ref, p1_ref, c_ref, h_ref, wl_ref, bl_ref,
                wr_ref, o_ref):
    agg = _mean_agg(p0_ref, p1_ref, c_ref)
    o_ref[...] = jnp.maximum(
        jnp.dot(agg, wl_ref[...], preferred_element_type=jnp.float32)
        + bl_ref[...]
        + jnp.dot(h_ref[...], wr_ref[...], preferred_element_type=jnp.float32),
        0.0)


def _layer_final_body(p0_ref, p1_ref, c_ref, h_ref, wl_ref, bl_ref,
                      wr_ref, w1_ref, b1_ref, o_ref):
    agg = _mean_agg(p0_ref, p1_ref, c_ref)
    h2 = jnp.maximum(
        jnp.dot(agg, wl_ref[...], preferred_element_type=jnp.float32)
        + bl_ref[...]
        + jnp.dot(h_ref[...], wr_ref[...], preferred_element_type=jnp.float32),
        0.0)
    o_ref[...] = (jnp.dot(h2, w1_ref[...], preferred_element_type=jnp.float32)
                  + b1_ref[...])


_row_spec = pl.BlockSpec((TBLK, D), lambda i: (i, 0))
_p0_spec = pl.BlockSpec((1, TBLK, D), lambda i: (0, i, 0))
_p1_spec = pl.BlockSpec((1, TBLK, D), lambda i: (1, i, 0))
_w_spec = pl.BlockSpec((D, D), lambda i: (0, 0))
_b_spec = pl.BlockSpec((1, D), lambda i: (0, 0))
_c_spec = pl.BlockSpec((2, TBLK // 128, 128), lambda i: (0, i, 0))
_out_pad = jax.ShapeDtypeStruct((NPAD, D), jnp.float32)
_out_n = jax.ShapeDtypeStruct((N, D), jnp.float32)
_grid = (NPAD // TBLK,)

_t0 = pl.pallas_call(
    _t0_body, grid=_grid,
    in_specs=[_row_spec, _w_spec, _b_spec],
    out_specs=_row_spec, out_shape=_out_pad)

_layer = pl.pallas_call(
    _layer_body, grid=_grid,
    in_specs=[_p0_spec, _p1_spec, _c_spec, _row_spec,
              _w_spec, _b_spec, _w_spec],
    out_specs=_row_spec, out_shape=_out_pad)

_layer_final = pl.pallas_call(
    _layer_final_body, grid=_grid,
    in_specs=[_p0_spec, _p1_spec, _c_spec, _row_spec,
              _w_spec, _b_spec, _w_spec, _w_spec, _b_spec],
    out_specs=_row_spec, out_shape=_out_n)


def kernel(x, edge_index, W0, b0, Wl1, bl1, Wr1, Wl2, bl2, Wr2, W1, b1):
    eflat = edge_index.reshape(2 * E)
    b0r = b0.reshape(1, D)
    bl1r = bl1.reshape(1, D)
    bl2r = bl2.reshape(1, D)
    b1r = b1.reshape(1, D)

    h0 = _t0(x, W0, b0r)
    part1, cnt = _seg_with_counts(h0, eflat)
    cnt3 = cnt.reshape(NC, NPAD // 128, 128)
    h1 = _layer(part1, part1, cnt3, h0, Wl1, bl1r, Wr1)
    (part2,) = _seg_no_counts(h1, eflat)
    out = _layer_final(part2, part2, cnt3, h1, Wl2, bl2r, Wr2, W1, b1r)
    return out


# speedup vs baseline: 1.3353x; 1.0141x over previous
import functools

import jax
import jax.numpy as jnp
from jax import lax
from jax.experimental import pallas as pl
from jax.experimental.pallas import tpu as pltpu
import jax.experimental.pallas.tpu_sc as plsc

N = 10000
D = 128
E = 320000

NC = 2
NS = 16
NW = NC * NS
EPW = E // NW
CHUNK = 96
NCHUNK = 104
TAIL = EPW - NCHUNK * CHUNK
NPAD = 10240
RPT = NPAD // NS
CPT = NPAD // NS


def _seg_body(with_counts, h_hbm, e_hbm, part_hbm, *rest):
    if with_counts:
        (cnt_hbm, src_v, dstc0, dstc1, dstct, rbuf0, rbuf1, tbuf,
         acc_s, cnt_s, ones_v, ones_t, cbuf,
         gsem0, gsem1, dsem0, dsem1) = rest
    else:
        (src_v, dstc0, dstc1, dstct, rbuf0, rbuf1, tbuf, acc_s,
         gsem0, gsem1, dsem0, dsem1) = rest
    c = lax.axis_index("c")
    s = lax.axis_index("s")
    w = c * NS + s
    dbase = E + w * EPW

    pltpu.sync_copy(e_hbm.at[pl.ds(w * EPW, EPW)], src_v)

    def _zrow(i, carry):
        def _zlane(j, carry2):
            rbuf0[i, pl.ds(j * 16, 16)] = jnp.zeros((16,), jnp.float32)
            return carry2
        return lax.fori_loop(0, D // 16, _zlane, carry)
    lax.fori_loop(0, CHUNK, _zrow, 0)
    zoffs = [(t * CHUNK, CHUNK) for t in range(RPT // CHUNK)]
    zoffs.append(((RPT // CHUNK) * CHUNK, RPT - (RPT // CHUNK) * CHUNK))
    for o, sz in zoffs:
        pltpu.sync_copy(rbuf0.at[pl.ds(0, sz), :],
                        acc_s.at[pl.ds(s * RPT + o, sz), :])

    if with_counts:
        def _zc(i, carry):
            cbuf[pl.ds(i * 16, 16)] = jnp.zeros((16,), jnp.float32)
            return carry
        lax.fori_loop(0, CPT // 16, _zc, 0)
        pltpu.sync_copy(cbuf, cnt_s.at[pl.ds(s * CPT, CPT)])
        for j in range(CHUNK // 16):
            ones_v[pl.ds(j * 16, 16)] = jnp.ones((16,), jnp.float32)
        ones_t[pl.ds(0, 16)] = jnp.ones((16,), jnp.float32)

    plsc.subcore_barrier()

    def _start(i, rb, dc, gsem, dsem):
        pltpu.async_copy(h_hbm.at[src_v.at[pl.ds(i * CHUNK, CHUNK)]], rb,
                         gsem)
        pltpu.async_copy(e_hbm.at[pl.ds(dbase + i * CHUNK, CHUNK)], dc, dsem)

    def _consume(i, rb, dc, gsem, dsem, prefetch_i):
        pltpu.make_async_copy(h_hbm.at[src_v.at[pl.ds(0, CHUNK)]], rb,
                              gsem).wait()
        pltpu.make_async_copy(e_hbm.at[pl.ds(0, CHUNK)], dc, dsem).wait()
        pltpu.sync_copy(rb, acc_s.at[dc], add=True)
        if with_counts:
            pltpu.sync_copy(ones_v, cnt_s.at[dc], add=True)

        @pl.when(prefetch_i < NCHUNK)
        def _():
            _start(prefetch_i, rb, dc, gsem, dsem)

    _start(0, rbuf0, dstc0, gsem0, dsem0)
    _start(1, rbuf1, dstc1, gsem1, dsem1)

    def _edge2(k, carry):
        i = 2 * k
        _consume(i, rbuf0, dstc0, gsem0, dsem0, i + 2)
        _consume(i + 1, rbuf1, dstc1, gsem1, dsem1, i + 3)
        return carry
    lax.fori_loop(0, NCHUNK // 2, _edge2, 0)

    pltpu.async_copy(h_hbm.at[src_v.at[pl.ds(NCHUNK * CHUNK, TAIL)]], tbuf,
                     gsem0)
    pltpu.sync_copy(e_hbm.at[pl.ds(dbase + NCHUNK * CHUNK, TAIL)], dstct)
    pltpu.make_async_copy(h_hbm.at[src_v.at[pl.ds(0, TAIL)]], tbuf,
                          gsem0).wait()
    pltpu.sync_copy(tbuf, acc_s.at[dstct], add=True)
    if with_counts:
        pltpu.sync_copy(ones_t, cnt_s.at[dstct], add=True)

    plsc.subcore_barrier()

    for o, sz in zoffs:
        pltpu.sync_copy(acc_s.at[pl.ds(s * RPT + o, sz), :],
                        rbuf0.at[pl.ds(0, sz), :])
        pltpu.sync_copy(rbuf0.at[pl.ds(0, sz), :],
                        part_hbm.at[c, pl.ds(s * RPT + o, sz), :])
    if with_counts:
        pltpu.sync_copy(cnt_s.at[pl.ds(s * CPT, CPT)], cbuf)
        pltpu.sync_copy(cbuf, cnt_hbm.at[pl.ds(c * NPAD + s * CPT, CPT)])


def _make_seg(with_counts):
    out_type = [jax.ShapeDtypeStruct((NC, NPAD, D), jnp.float32)]
    scratch = [
        pltpu.VMEM((EPW,), jnp.int32),
        pltpu.VMEM((CHUNK,), jnp.int32),
        pltpu.VMEM((CHUNK,), jnp.int32),
        pltpu.VMEM((TAIL,), jnp.int32),
        pltpu.VMEM((CHUNK, D), jnp.float32),
        pltpu.VMEM((CHUNK, D), jnp.float32),
        pltpu.VMEM((TAIL, D), jnp.float32),
        pltpu.VMEM_SHARED((NPAD, D), jnp.float32),
    ]
    if with_counts:
        out_type.append(jax.ShapeDtypeStruct((NC * NPAD,), jnp.float32))
        scratch += [
            pltpu.VMEM_SHARED((NPAD,), jnp.float32),
            pltpu.VMEM((CHUNK,), jnp.float32),
            pltpu.VMEM((TAIL,), jnp.float32),
            pltpu.VMEM((CPT,), jnp.float32),
        ]
    scratch += [pltpu.SemaphoreType.DMA] * 4
    return pl.kernel(
        functools.partial(_seg_body, with_counts),
        out_type=out_type,
        mesh=plsc.VectorSubcoreMesh(core_axis_name="c", subcore_axis_name="s"),
        scratch_types=scratch,
    )


_seg_with_counts = _make_seg(True)
_seg_no_counts = _make_seg(False)


TBLK = 1024


def _t0_body(x_ref, w_ref, b_ref, o_ref):
    o_ref[...] = jnp.maximum(
        jnp.dot(x_ref[...], w_ref[...], preferred_element_type=jnp.float32)
        + b_ref[...], 0.0)


def _mean_agg(p0_ref, p1_ref, c_ref):
    cs = c_ref[0] + c_ref[1]
    cols = []
    for j in range(TBLK // 128):
        m = jnp.broadcast_to(cs[j][None, :], (128, 128))
        cols.append(m.T)
    cnt = jnp.maximum(jnp.concatenate(cols, axis=0), 1.0)
    return (p0_ref[0] + p1_ref[0]) / cnt


def _layer_body(p0_ref, p1_ref, c_ref, h_ref, wl_ref, bl_ref,
                wr_ref, o_ref):
    agg = _mean_agg(p0_ref, p1_ref, c_ref)
    o_ref[...] = jnp.maximum(
        jnp.dot(agg, wl_ref[...], preferred_element_type=jnp.float32)
        + bl_ref[...]
        + jnp.dot(h_ref[...], wr_ref[...], preferred_element_type=jnp.float32),
        0.0)


def _layer_final_body(p0_ref, p1_ref, c_ref, h_ref, wl_ref, bl_ref,
                      wr_ref, w1_ref, b1_ref, o_ref):
    agg = _mean_agg(p0_ref, p1_ref, c_ref)
    h2 = jnp.maximum(
        jnp.dot(agg, wl_ref[...], preferred_element_type=jnp.float32)
        + bl_ref[...]
        + jnp.dot(h_ref[...], wr_ref[...], preferred_element_type=jnp.float32),
        0.0)
    o_ref[...] = (jnp.dot(h2, w1_ref[...], preferred_element_type=jnp.float32)
                  + b1_ref[...])


_row_spec = pl.BlockSpec((TBLK, D), lambda i: (i, 0))
_p0_spec = pl.BlockSpec((1, TBLK, D), lambda i: (0, i, 0))
_p1_spec = pl.BlockSpec((1, TBLK, D), lambda i: (1, i, 0))
_w_spec = pl.BlockSpec((D, D), lambda i: (0, 0))
_b_spec = pl.BlockSpec((1, D), lambda i: (0, 0))
_c_spec = pl.BlockSpec((2, TBLK // 128, 128), lambda i: (0, i, 0))
_out_pad = jax.ShapeDtypeStruct((NPAD, D), jnp.float32)
_out_n = jax.ShapeDtypeStruct((N, D), jnp.float32)
_grid = (NPAD // TBLK,)

_t0 = pl.pallas_call(
    _t0_body, grid=_grid,
    in_specs=[_row_spec, _w_spec, _b_spec],
    out_specs=_row_spec, out_shape=_out_pad)

_layer = pl.pallas_call(
    _layer_body, grid=_grid,
    in_specs=[_p0_spec, _p1_spec, _c_spec, _row_spec,
              _w_spec, _b_spec, _w_spec],
    out_specs=_row_spec, out_shape=_out_pad)

_layer_final = pl.pallas_call(
    _layer_final_body, grid=_grid,
    in_specs=[_p0_spec, _p1_spec, _c_spec, _row_spec,
              _w_spec, _b_spec, _w_spec, _w_spec, _b_spec],
    out_specs=_row_spec, out_shape=_out_n)


def kernel(x, edge_index, W0, b0, Wl1, bl1, Wr1, Wl2, bl2, Wr2, W1, b1):
    eflat = edge_index.reshape(2 * E)
    b0r = b0.reshape(1, D)
    bl1r = bl1.reshape(1, D)
    bl2r = bl2.reshape(1, D)
    b1r = b1.reshape(1, D)

    h0 = _t0(x, W0, b0r)
    part1, cnt = _seg_with_counts(h0, eflat)
    cnt3 = cnt.reshape(NC, NPAD // 128, 128)
    h1 = _layer(part1, part1, cnt3, h0, Wl1, bl1r, Wr1)
    (part2,) = _seg_no_counts(h1, eflat)
    out = _layer_final(part2, part2, cnt3, h1, Wl2, bl2r, Wr2, W1, b1r)
    return out
